# baseline (speedup 1.0000x reference)
import functools
import jax
import jax.numpy as jnp
from jax import lax
from jax.experimental import pallas as pl
from jax.experimental.pallas import tpu as pltpu
from jax.experimental.pallas import tpu_sc as plsc

_B = 128
_V = 100000
_K = 10
_DELTA = 0.01
_NC = 2
_NS = 16
_NW = _NC * _NS
_NTILES = 782
_NFULL_A = 391
_NFULL_B = 390
_SEG = 98
_NSEG = 4
_TAILC = _V - 781 * 128
_BIG = 2147483647

_mesh = plsc.VectorSubcoreMesh(
    core_axis_name="c", subcore_axis_name="s", num_cores=_NC, num_subcores=_NS
)


@functools.partial(
    pl.kernel,
    out_type=jax.ShapeDtypeStruct((_NW * 64,), jnp.float32),
    mesh=_mesh,
    scratch_types=[
        pltpu.VMEM((_SEG, 8, 128), jnp.float32),
        pltpu.VMEM((112 * 16,), jnp.float32),
        pltpu.VMEM((112 * 16,), jnp.int32),
        pltpu.VMEM((7 * 16,), jnp.float32),
        pltpu.VMEM((7 * 16,), jnp.int32),
        pltpu.VMEM((16, 8, 128), jnp.float32),
        pltpu.VMEM((512,), jnp.float32),
        pltpu.VMEM((512,), jnp.int32),
        pltpu.VMEM((16,), jnp.float32),
        pltpu.VMEM((528,), jnp.float32),
        pltpu.VMEM((512,), jnp.int32),
        pltpu.VMEM((528,), jnp.float32),
        pltpu.VMEM((512,), jnp.int32),
        pltpu.VMEM((16,), jnp.float32),
        pltpu.VMEM((16,), jnp.int32),
        pltpu.VMEM((64,), jnp.float32),
        pltpu.VMEM_SHARED((16 * 544,), jnp.float32),
        pltpu.VMEM_SHARED((16 * 512,), jnp.int32),
        pltpu.SemaphoreType.DMA,
    ],
)
def _sc_loss(scores_hbm, labels_hbm, stail_hbm, ltail_hbm, out_hbm, slab,
             gmval, gme, sgval, sge, lbuf, candv, candi, candc, rdav, rdai,
             rdbv, rdbi, tmpv, tmpi, ovec, shv, shi, sem):
    cid = lax.axis_index("c")
    sid = lax.axis_index("s")
    wid = cid * _NS + sid
    iota = lax.iota(jnp.int32, 16)
    ninf = jnp.float32(-jnp.inf)
    ninf_vec = jnp.full((16,), ninf, jnp.float32)
    big_vec = jnp.full((16,), _BIG, jnp.int32)

    is_b = sid & 1
    rb8 = (cid * 8 + (sid >> 1)) * 8
    nfull = jnp.where(is_b == 1, jnp.int32(_NFULL_B), jnp.int32(_NFULL_A))
    half0 = jnp.where(is_b == 1, jnp.int32(_NFULL_A), jnp.int32(0))

    _dnums = lax.GatherDimensionNumbers(
        offset_dims=(), collapsed_slice_dims=(0,), start_index_map=(0,))

    def _gather16(v, idx):
        return lax.gather(v, idx[:, None], _dnums, slice_sizes=(1,),
                          mode=lax.GatherScatterMode.PROMISE_IN_BOUNDS)

    def _bfly(v, op):
        for sh in (8, 4, 2, 1):
            v = op(v, _gather16(v, iota ^ sh))
        return v[0]

    def _rmax_f(v):
        return _bfly(v, jnp.maximum)

    def _rmin_i(v):
        return _bfly(v, jnp.minimum)

    def _rsum(v):
        return _bfly(v, lax.add)

    def pair_tree(vs, ts):
        vs, ts = list(vs), list(ts)
        while len(vs) > 1:
            nv, nt = [], []
            for u in range(0, len(vs) - 1, 2):
                gt = vs[u + 1] > vs[u]
                nv.append(jnp.where(gt, vs[u + 1], vs[u]))
                nt.append(jnp.where(gt, ts[u + 1], ts[u]))
            if len(vs) % 2:
                nv.append(vs[-1])
                nt.append(ts[-1])
            vs, ts = nv, nt
        return vs[0], ts[0]

    def tile_tree(j2, r):
        vs = [slab[j2, r, pl.ds(t * 16, 16)] for t in range(8)]
        ts = [jnp.full((16,), t, jnp.int32) for t in range(8)]
        while len(vs) > 1:
            nv, nt = [], []
            for u in range(0, len(vs), 2):
                gt = vs[u + 1] > vs[u]
                nv.append(jnp.where(gt, vs[u + 1], vs[u]))
                nt.append(jnp.where(gt, ts[u + 1], ts[u]))
            vs, ts = nv, nt
        return vs[0], ts[0] * 16 + iota

    def build_hier(r, ntt, count_finite):
        def bpad(j, _):
            gmval[pl.ds(j * 16, 16)] = ninf_vec
            gme[pl.ds(j * 16, 16)] = big_vec
            return 0
        lax.fori_loop(ntt, 112, bpad, 0)

        def bt(j2, cacc):
            gv, gt3 = tile_tree(j2, r)
            gmval[pl.ds(j2 * 16, 16)] = gv
            gme[pl.ds(j2 * 16, 16)] = j2 * 128 + gt3
            if count_finite:
                for t in range(8):
                    d = slab[j2, r, pl.ds(t * 16, 16)]
                    cacc = cacc + jnp.where(d > ninf, 1.0, 0.0)
            return cacc
        cacc = lax.fori_loop(0, ntt, bt, jnp.zeros((16,), jnp.float32))

        def bs(s, _):
            vv = [gmval[pl.ds(s * 256 + tt * 16, 16)] for tt in range(16)]
            ii = [gme[pl.ds(s * 256 + tt * 16, 16)] for tt in range(16)]
            sv, si = pair_tree(vv, ii)
            sgval[pl.ds(s * 16, 16)] = sv
            sge[pl.ds(s * 16, 16)] = si
            return 0
        lax.fori_loop(0, 7, bs, 0)
        return cacc

    def extract10(r, ntt, tile0):
        def round_body(rnd, carry):
            vals, idxs = carry
            m, mi = pair_tree(
                [sgval[pl.ds(s * 16, 16)] for s in range(7)],
                [sge[pl.ds(s * 16, 16)] for s in range(7)])
            mval = _rmax_f(m)
            ei = jnp.where(m == mval, mi, _BIG)
            e = jnp.minimum(_rmin_i(ei), ntt * 128 - 1)
            g = jnp.minimum(tile0 * 128 + e, jnp.int32(_V - 1))
            vals = jnp.where(iota == rnd, mval, vals)
            idxs = jnp.where(iota == rnd, g, idxs)
            j2 = e >> 7
            d = slab[j2, r, pl.ds(((e >> 4) & 7) * 16, 16)]
            slab[j2, r, pl.ds(((e >> 4) & 7) * 16, 16)] = (
                jnp.where(iota == (e & 15), ninf, d))
            gv, gt3 = tile_tree(j2, r)
            gmval[pl.ds(j2 * 16, 16)] = gv
            gme[pl.ds(j2 * 16, 16)] = j2 * 128 + gt3
            s2 = j2 >> 4
            sv, si = pair_tree(
                [gmval[pl.ds(s2 * 256 + tt * 16, 16)] for tt in range(16)],
                [gme[pl.ds(s2 * 256 + tt * 16, 16)] for tt in range(16)])
            sgval[pl.ds(s2 * 16, 16)] = sv
            sge[pl.ds(s2 * 16, 16)] = si
            return vals, idxs
        return lax.fori_loop(0, _K, round_body,
                             (ninf_vec, jnp.zeros((16,), jnp.int32)))

    def restore10(r, tile0, vals, idxs):
        def rbody(r2, _):
            mval = _rmax_f(jnp.where(iota == r2, vals, ninf_vec))
            g = _rmin_i(jnp.where(iota == r2, idxs, big_vec))
            e = g - tile0 * 128
            j2 = e >> 7
            d = slab[j2, r, pl.ds(((e >> 4) & 7) * 16, 16)]
            slab[j2, r, pl.ds(((e >> 4) & 7) * 16, 16)] = (
                jnp.where(iota == (e & 15), mval, d))
            return 0
        lax.fori_loop(0, _K, rbody, 0)

    def rank_merge_refs(rv, ri, sv, si):
        cr = jnp.zeros((16,), jnp.int32)
        cs = jnp.zeros((16,), jnp.int32)
        for j in range(_K):
            cr = cr + jnp.where(sv[j] > rv, 1, 0)
            cs = cs + jnp.where(rv[j] >= sv, 1, 0)
        rank_r = iota + cr
        rank_s = iota + cs
        src_a = jnp.zeros((16,), jnp.int32)
        has_a = jnp.zeros((16,), jnp.int32)
        src_b = jnp.zeros((16,), jnp.int32)
        for j in range(_K):
            ra = rank_r[j]
            src_a = src_a + jnp.where(iota == ra, j, 0)
            has_a = has_a + jnp.where(iota == ra, 1, 0)
            rb = rank_s[j]
            src_b = src_b + jnp.where(iota == rb, j, 0)
        va = _gather16(rv, src_a)
        ia = _gather16(ri, src_a)
        vb = _gather16(sv, src_b)
        ib = _gather16(si, src_b)
        mv = jnp.where(has_a > 0, va, vb)
        mi = jnp.where(has_a > 0, ia, ib)
        return mv, mi

    def seg_params(seg):
        tile0_rel = seg * _SEG
        ntf = jnp.minimum(jnp.int32(_SEG), nfull - tile0_rel)
        has_tail = (is_b == 1) & (seg == 3)
        ntt = ntf + jnp.where(has_tail, 1, 0)
        tile0 = half0 + tile0_rel
        return tile0, ntf, ntt, has_tail

    def cinit(i, _):
        candv[pl.ds(i * 16, 16)] = ninf_vec
        candi[pl.ds(i * 16, 16)] = big_vec
        return 0
    lax.fori_loop(0, 32, cinit, 0)
    candc[...] = jnp.zeros((16,), jnp.float32)

    def seg_body(seg, _):
        tile0, ntf, ntt, has_tail = seg_params(seg)

        def siss(j, _):
            pltpu.make_async_copy(
                scores_hbm.at[pl.ds(pl.multiple_of(rb8, 8), 8), pl.ds((tile0 + j) * 128, 128)],
                slab.at[j], sem).start()
            return 0
        lax.fori_loop(0, ntf, siss, 0)

        @pl.when(has_tail)
        def _():
            pltpu.make_async_copy(
                stail_hbm.at[pl.ds(pl.multiple_of(rb8, 8), 8)], slab.at[ntf], sem).start()

        def sdrain(j, _):
            pltpu.make_async_copy(
                scores_hbm.at[pl.ds(pl.multiple_of(rb8, 8), 8), pl.ds((tile0 + j) * 128, 128)],
                slab.at[j], sem).wait()
            return 0
        lax.fori_loop(0, ntf, sdrain, 0)

        @pl.when(has_tail)
        def _():
            pltpu.make_async_copy(
                stail_hbm.at[pl.ds(pl.multiple_of(rb8, 8), 8)], slab.at[ntf], sem).wait()

        def p1(r, _):
            build_hier(r, ntt, False)
            vals, idxs = extract10(r, ntt, tile0)
            restore10(r, tile0, vals, idxs)
            rv = candv[pl.ds(r * 64, 16)]
            ri = candi[pl.ds(r * 64, 16)]
            mv, mi = rank_merge_refs(rv, ri, vals, idxs)
            candv[pl.ds(r * 64, 16)] = mv
            candi[pl.ds(r * 64, 16)] = mi
            return 0
        lax.fori_loop(0, 8, p1, 0)

        nbatch = (ntf + 7) >> 3

        def liss(b, _):
            def ibody(jj, _):
                j = b * 8 + jj
                pltpu.make_async_copy(
                    labels_hbm.at[pl.ds(pl.multiple_of(rb8, 8), 8),
                                  pl.ds((tile0 + j) * 128, 128)],
                    lbuf.at[(b & 1) * 8 + jj], sem).start()
                return 0
            lax.fori_loop(0, jnp.minimum(jnp.int32(8), ntf - b * 8), ibody, 0)
            return 0

        def ldrain_mask(b, _):
            nb = jnp.minimum(jnp.int32(8), ntf - b * 8)
            def dbody(jj, _):
                j = b * 8 + jj
                pltpu.make_async_copy(
                    labels_hbm.at[pl.ds(pl.multiple_of(rb8, 8), 8),
                                  pl.ds((tile0 + j) * 128, 128)],
                    lbuf.at[(b & 1) * 8 + jj], sem).wait()
                return 0
            lax.fori_loop(0, nb, dbody, 0)
            @pl.when(b + 1 < nbatch)
            def _():
                liss(b + 1, 0)
            def mbody(p, _):
                jj = p >> 3
                rr = p & 7
                j = b * 8 + jj
                for t in range(8):
                    lab = lbuf[(b & 1) * 8 + jj, rr, pl.ds(t * 16, 16)]
                    d = slab[j, rr, pl.ds(t * 16, 16)]
                    slab[j, rr, pl.ds(t * 16, 16)] = (
                        jnp.where(lab == 1.0, d, ninf))
                return 0
            lax.fori_loop(0, nb * 8, mbody, 0)
            return 0

        liss(0, 0)
        lax.fori_loop(0, nbatch, ldrain_mask, 0)

        @pl.when(has_tail)
        def _():
            pltpu.make_async_copy(
                ltail_hbm.at[pl.ds(pl.multiple_of(rb8, 8), 8)], lbuf.at[0], sem).start()
            pltpu.make_async_copy(
                ltail_hbm.at[pl.ds(pl.multiple_of(rb8, 8), 8)], lbuf.at[0], sem).wait()
            for rr in range(8):
                for t in range(8):
                    lab = lbuf[0, rr, pl.ds(t * 16, 16)]
                    d = slab[ntf, rr, pl.ds(t * 16, 16)]
                    slab[ntf, rr, pl.ds(t * 16, 16)] = (
                        jnp.where(lab == 1.0, d, ninf))

        def p2(r, _):
            cacc = build_hier(r, ntt, True)
            cnt_r = _rsum(cacc)
            candc[...] = candc[...] + jnp.where(iota == r, cnt_r, 0.0)
            vals, idxs = extract10(r, ntt, tile0)
            rv = candv[pl.ds(r * 64 + 16, 16)]
            ri = candi[pl.ds(r * 64 + 16, 16)]
            mv, mi = rank_merge_refs(rv, ri, vals, idxs)
            candv[pl.ds(r * 64 + 16, 16)] = mv
            candi[pl.ds(r * 64 + 16, 16)] = mi
            return 0
        lax.fori_loop(0, 8, p2, 0)
        return 0

    lax.fori_loop(0, _NSEG, seg_body, 0)

    _mo = lambda x: pl.multiple_of(x, 8)
    pltpu.sync_copy(candv, shv.at[pl.ds(_mo(sid * 544), 512)])
    pltpu.sync_copy(candc, shv.at[pl.ds(_mo(sid * 544 + 512), 16)])
    pltpu.sync_copy(candi, shi.at[pl.ds(_mo(sid * 512), 512)])
    plsc.subcore_barrier()
    pair = sid & ~1
    pltpu.sync_copy(shv.at[pl.ds(_mo(pair * 544), 528)], rdav)
    pltpu.sync_copy(shi.at[pl.ds(_mo(pair * 512), 512)], rdai)
    pltpu.sync_copy(shv.at[pl.ds(_mo((pair + 1) * 544), 528)], rdbv)
    pltpu.sync_copy(shi.at[pl.ds(_mo((pair + 1) * 512), 512)], rdbi)

    cnt_both = rdav[pl.ds(512, 16)] + rdbv[pl.ds(512, 16)]

    def loss_row(r, acc):
        a1, b1, a2, b2 = acc
        pv, pi = rank_merge_refs(rdav[pl.ds(r * 64, 16)],
                                 rdai[pl.ds(r * 64, 16)],
                                 rdbv[pl.ds(r * 64, 16)],
                                 rdbi[pl.ds(r * 64, 16)])
        tv, ti = rank_merge_refs(rdav[pl.ds(r * 64 + 16, 16)],
                                 rdai[pl.ds(r * 64 + 16, 16)],
                                 rdbv[pl.ds(r * 64 + 16, 16)],
                                 rdbi[pl.ds(r * 64 + 16, 16)])
        num_true = _rsum(jnp.where(iota == r, cnt_both, 0.0))
        nv = jnp.minimum(jnp.int32(_K), num_true.astype(jnp.int32))
        pnt_i = jnp.where(iota < _K, jnp.int32(1), jnp.int32(0))
        for j in range(_K):
            jvec = iota * 0 + j
            keep_i = (jnp.where(pi != ti[j], jnp.int32(1), jnp.int32(0))
                      + jnp.where(jvec >= nv, jnp.int32(1), jnp.int32(0)))
            pnt_i = jnp.where(keep_i > 0, pnt_i, jnp.int32(0))
        npnt = _rsum(pnt_i)
        s1 = jnp.float32(0.0)
        s2 = jnp.float32(0.0)
        for i in range(_K):
            ti_v = tv[i]
            ivec = iota * 0 + i
            one = jnp.int32(1)
            zero = jnp.int32(0)
            validi_i = jnp.where(ivec < nv, one, zero)
            tvalid_i = jnp.where(iota < nv, one, zero)
            gti_i = jnp.where(iota > i, one, zero)
            c1 = jnp.maximum(jnp.float32(_DELTA) - (ti_v - tv), 0.0)
            m1_i = gti_i * tvalid_i * validi_i
            s1 = s1 + _rsum(jnp.where(m1_i > 0, c1, 0.0))
            c2 = jnp.maximum(jnp.float32(_DELTA) - (ti_v - pv), 0.0)
            m2_i = pnt_i * validi_i
            s2 = s2 + _rsum(jnp.where(m2_i > 0, c2, 0.0))
        cnt1 = ((nv * (nv - 1)) >> 1).astype(jnp.float32)
        cnt2 = (nv * npnt).astype(jnp.float32)
        a1 = a1 + jnp.where(iota == r, s1, 0.0)
        b1 = b1 + jnp.where(iota == r, cnt1, 0.0)
        a2 = a2 + jnp.where(iota == r, s2, 0.0)
        b2 = b2 + jnp.where(iota == r, cnt2, 0.0)
        return a1, b1, a2, b2

    zf = jnp.zeros((16,), jnp.float32)
    a1, b1, a2, b2 = lax.fori_loop(0, 8, loss_row, (zf, zf, zf, zf))
    evenf = jnp.where(is_b == 0, jnp.float32(1.0), jnp.float32(0.0))
    ovec[pl.ds(0, 16)] = a1 * evenf
    ovec[pl.ds(16, 16)] = b1 * evenf
    ovec[pl.ds(32, 16)] = a2 * evenf
    ovec[pl.ds(48, 16)] = b2 * evenf
    pltpu.sync_copy(ovec, out_hbm.at[pl.ds(wid * 64, 64)])


def _red_body(x_ref, o_ref):
    x = x_ref[...]
    s1 = x[:, 0:16]
    c1 = x[:, 16:32]
    s2 = x[:, 32:48]
    c2 = x[:, 48:64]
    l = s1 / jnp.maximum(c1, 1.0) + s2 / jnp.maximum(c2, 1.0)
    o_ref[...] = (jnp.sum(l) * jnp.float32(1.0 / _B)).reshape(1, 1)


_tc_reduce = pl.pallas_call(
    _red_body,
    out_shape=jax.ShapeDtypeStruct((1, 1), jnp.float32),
)


@jax.jit
def kernel(scores, true_labels):
    stail = jnp.pad(scores[:, 781 * 128:], ((0, 0), (0, 128 - _TAILC)),
                    constant_values=-jnp.inf)
    ltail = jnp.pad(true_labels[:, 781 * 128:], ((0, 0), (0, 128 - _TAILC)),
                    constant_values=0.0)
    partial = _sc_loss(scores, true_labels, stail, ltail)
    return _tc_reduce(partial.reshape(_NW, 64))[0, 0]


# speedup vs baseline: 1.0035x; 1.0035x over previous
import functools
import jax
import jax.numpy as jnp
from jax import lax
from jax.experimental import pallas as pl
from jax.experimental.pallas import tpu as pltpu
from jax.experimental.pallas import tpu_sc as plsc

_B = 128
_V = 100000
_K = 10
_DELTA = 0.01
_NC = 2
_NS = 16
_NW = _NC * _NS
_NTILES = 782
_NFULL_A = 391
_NFULL_B = 390
_SEG = 98
_NSEG = 4
_TAILC = _V - 781 * 128
_BIG = 2147483647

_mesh = plsc.VectorSubcoreMesh(
    core_axis_name="c", subcore_axis_name="s", num_cores=_NC, num_subcores=_NS
)


@functools.partial(
    pl.kernel,
    out_type=jax.ShapeDtypeStruct((_NW * 64,), jnp.float32),
    mesh=_mesh,
    scratch_types=[
        pltpu.VMEM((_SEG, 8, 128), jnp.float32),
        pltpu.VMEM((112 * 16,), jnp.float32),
        pltpu.VMEM((112 * 16,), jnp.int32),
        pltpu.VMEM((7 * 16,), jnp.float32),
        pltpu.VMEM((7 * 16,), jnp.int32),
        pltpu.VMEM((16, 8, 128), jnp.float32),
        pltpu.VMEM((512,), jnp.float32),
        pltpu.VMEM((512,), jnp.int32),
        pltpu.VMEM((16,), jnp.float32),
        pltpu.VMEM((528,), jnp.float32),
        pltpu.VMEM((512,), jnp.int32),
        pltpu.VMEM((528,), jnp.float32),
        pltpu.VMEM((512,), jnp.int32),
        pltpu.VMEM((16,), jnp.float32),
        pltpu.VMEM((16,), jnp.int32),
        pltpu.VMEM((64,), jnp.float32),
        pltpu.VMEM_SHARED((16 * 544,), jnp.float32),
        pltpu.VMEM_SHARED((16 * 512,), jnp.int32),
        pltpu.SemaphoreType.DMA,
    ],
)
def _sc_loss(scores_hbm, labels_hbm, stail_hbm, ltail_hbm, out_hbm, slab,
             gmval, gme, sgval, sge, lbuf, candv, candi, candc, rdav, rdai,
             rdbv, rdbi, tmpv, tmpi, ovec, shv, shi, sem):
    cid = lax.axis_index("c")
    sid = lax.axis_index("s")
    wid = cid * _NS + sid
    iota = lax.iota(jnp.int32, 16)
    ninf = jnp.float32(-jnp.inf)
    ninf_vec = jnp.full((16,), ninf, jnp.float32)
    big_vec = jnp.full((16,), _BIG, jnp.int32)

    is_b = sid & 1
    rb8 = (cid * 8 + (sid >> 1)) * 8
    nfull = jnp.where(is_b == 1, jnp.int32(_NFULL_B), jnp.int32(_NFULL_A))
    half0 = jnp.where(is_b == 1, jnp.int32(_NFULL_A), jnp.int32(0))

    _dnums = lax.GatherDimensionNumbers(
        offset_dims=(), collapsed_slice_dims=(0,), start_index_map=(0,))

    def _gather16(v, idx):
        return lax.gather(v, idx[:, None], _dnums, slice_sizes=(1,),
                          mode=lax.GatherScatterMode.PROMISE_IN_BOUNDS)

    def _bfly(v, op):
        for sh in (8, 4, 2, 1):
            v = op(v, _gather16(v, iota ^ sh))
        return v[0]

    def _rmax_f(v):
        return _bfly(v, jnp.maximum)

    def _rmin_i(v):
        return _bfly(v, jnp.minimum)

    def _rsum(v):
        return _bfly(v, lax.add)

    def pair_tree(vs, ts):
        vs, ts = list(vs), list(ts)
        while len(vs) > 1:
            nv, nt = [], []
            for u in range(0, len(vs) - 1, 2):
                gt = vs[u + 1] > vs[u]
                nv.append(jnp.where(gt, vs[u + 1], vs[u]))
                nt.append(jnp.where(gt, ts[u + 1], ts[u]))
            if len(vs) % 2:
                nv.append(vs[-1])
                nt.append(ts[-1])
            vs, ts = nv, nt
        return vs[0], ts[0]

    def tile_tree(j2, r):
        vs = [slab[j2, r, pl.ds(t * 16, 16)] for t in range(8)]
        ts = [jnp.full((16,), t, jnp.int32) for t in range(8)]
        while len(vs) > 1:
            nv, nt = [], []
            for u in range(0, len(vs), 2):
                gt = vs[u + 1] > vs[u]
                nv.append(jnp.where(gt, vs[u + 1], vs[u]))
                nt.append(jnp.where(gt, ts[u + 1], ts[u]))
            vs, ts = nv, nt
        return vs[0], ts[0] * 16 + iota

    def build_hier(r, ntt, count_finite):
        def bpad(j, _):
            gmval[pl.ds(j * 16, 16)] = ninf_vec
            gme[pl.ds(j * 16, 16)] = big_vec
            return 0
        lax.fori_loop(ntt, 112, bpad, 0)

        def bt(j2, cacc):
            gv, gt3 = tile_tree(j2, r)
            gmval[pl.ds(j2 * 16, 16)] = gv
            gme[pl.ds(j2 * 16, 16)] = j2 * 128 + gt3
            if count_finite:
                for t in range(8):
                    d = slab[j2, r, pl.ds(t * 16, 16)]
                    cacc = cacc + jnp.where(d > ninf, 1.0, 0.0)
            return cacc
        cacc = lax.fori_loop(0, ntt, bt, jnp.zeros((16,), jnp.float32))

        def bs(s, _):
            vv = [gmval[pl.ds(s * 256 + tt * 16, 16)] for tt in range(16)]
            ii = [gme[pl.ds(s * 256 + tt * 16, 16)] for tt in range(16)]
            sv, si = pair_tree(vv, ii)
            sgval[pl.ds(s * 16, 16)] = sv
            sge[pl.ds(s * 16, 16)] = si
            return 0
        lax.fori_loop(0, 7, bs, 0)
        return cacc

    def extract10(r, ntt, tile0):
        def round_body(rnd, carry):
            vals, idxs = carry
            m, mi = pair_tree(
                [sgval[pl.ds(s * 16, 16)] for s in range(7)],
                [sge[pl.ds(s * 16, 16)] for s in range(7)])
            for sh in (8, 4, 2, 1):
                gv_ = _gather16(m, iota ^ sh)
                ge_ = _gather16(mi, iota ^ sh)
                upd = (gv_ > m) | ((gv_ == m) & (ge_ < mi))
                m = jnp.where(upd, gv_, m)
                mi = jnp.where(upd, ge_, mi)
            mval = m[0]
            e = jnp.minimum(mi[0], ntt * 128 - 1)
            g = jnp.minimum(tile0 * 128 + e, jnp.int32(_V - 1))
            vals = jnp.where(iota == rnd, mval, vals)
            idxs = jnp.where(iota == rnd, g, idxs)
            j2 = e >> 7
            d = slab[j2, r, pl.ds(((e >> 4) & 7) * 16, 16)]
            slab[j2, r, pl.ds(((e >> 4) & 7) * 16, 16)] = (
                jnp.where(iota == (e & 15), ninf, d))
            gv, gt3 = tile_tree(j2, r)
            gmval[pl.ds(j2 * 16, 16)] = gv
            gme[pl.ds(j2 * 16, 16)] = j2 * 128 + gt3
            s2 = j2 >> 4
            sv, si = pair_tree(
                [gmval[pl.ds(s2 * 256 + tt * 16, 16)] for tt in range(16)],
                [gme[pl.ds(s2 * 256 + tt * 16, 16)] for tt in range(16)])
            sgval[pl.ds(s2 * 16, 16)] = sv
            sge[pl.ds(s2 * 16, 16)] = si
            return vals, idxs
        return lax.fori_loop(0, _K, round_body,
                             (ninf_vec, jnp.zeros((16,), jnp.int32)))

    def restore10(r, tile0, vals, idxs):
        def rbody(r2, _):
            mval = _rmax_f(jnp.where(iota == r2, vals, ninf_vec))
            g = _rmin_i(jnp.where(iota == r2, idxs, big_vec))
            e = g - tile0 * 128
            j2 = e >> 7
            d = slab[j2, r, pl.ds(((e >> 4) & 7) * 16, 16)]
            slab[j2, r, pl.ds(((e >> 4) & 7) * 16, 16)] = (
                jnp.where(iota == (e & 15), mval, d))
            return 0
        lax.fori_loop(0, _K, rbody, 0)

    def rank_merge_refs(rv, ri, sv, si):
        cr = jnp.zeros((16,), jnp.int32)
        cs = jnp.zeros((16,), jnp.int32)
        for j in range(_K):
            cr = cr + jnp.where(sv[j] > rv, 1, 0)
            cs = cs + jnp.where(rv[j] >= sv, 1, 0)
        rank_r = iota + cr
        rank_s = iota + cs
        src_a = jnp.zeros((16,), jnp.int32)
        has_a = jnp.zeros((16,), jnp.int32)
        src_b = jnp.zeros((16,), jnp.int32)
        for j in range(_K):
            ra = rank_r[j]
            src_a = src_a + jnp.where(iota == ra, j, 0)
            has_a = has_a + jnp.where(iota == ra, 1, 0)
            rb = rank_s[j]
            src_b = src_b + jnp.where(iota == rb, j, 0)
        va = _gather16(rv, src_a)
        ia = _gather16(ri, src_a)
        vb = _gather16(sv, src_b)
        ib = _gather16(si, src_b)
        mv = jnp.where(has_a > 0, va, vb)
        mi = jnp.where(has_a > 0, ia, ib)
        return mv, mi

    def seg_params(seg):
        tile0_rel = seg * _SEG
        ntf = jnp.minimum(jnp.int32(_SEG), nfull - tile0_rel)
        has_tail = (is_b == 1) & (seg == 3)
        ntt = ntf + jnp.where(has_tail, 1, 0)
        tile0 = half0 + tile0_rel
        return tile0, ntf, ntt, has_tail

    def cinit(i, _):
        candv[pl.ds(i * 16, 16)] = ninf_vec
        candi[pl.ds(i * 16, 16)] = big_vec
        return 0
    lax.fori_loop(0, 32, cinit, 0)
    candc[...] = jnp.zeros((16,), jnp.float32)

    def seg_body(seg, _):
        tile0, ntf, ntt, has_tail = seg_params(seg)

        def siss(j, _):
            pltpu.make_async_copy(
                scores_hbm.at[pl.ds(pl.multiple_of(rb8, 8), 8), pl.ds((tile0 + j) * 128, 128)],
                slab.at[j], sem).start()
            return 0
        lax.fori_loop(0, ntf, siss, 0)

        @pl.when(has_tail)
        def _():
            pltpu.make_async_copy(
                stail_hbm.at[pl.ds(pl.multiple_of(rb8, 8), 8)], slab.at[ntf], sem).start()

        def sdrain(j, _):
            pltpu.make_async_copy(
                scores_hbm.at[pl.ds(pl.multiple_of(rb8, 8), 8), pl.ds((tile0 + j) * 128, 128)],
                slab.at[j], sem).wait()
            return 0
        lax.fori_loop(0, ntf, sdrain, 0)

        @pl.when(has_tail)
        def _():
            pltpu.make_async_copy(
                stail_hbm.at[pl.ds(pl.multiple_of(rb8, 8), 8)], slab.at[ntf], sem).wait()

        def p1(r, _):
            build_hier(r, ntt, False)
            vals, idxs = extract10(r, ntt, tile0)
            restore10(r, tile0, vals, idxs)
            rv = candv[pl.ds(r * 64, 16)]
            ri = candi[pl.ds(r * 64, 16)]
            mv, mi = rank_merge_refs(rv, ri, vals, idxs)
            candv[pl.ds(r * 64, 16)] = mv
            candi[pl.ds(r * 64, 16)] = mi
            return 0
        lax.fori_loop(0, 8, p1, 0)

        nbatch = (ntf + 7) >> 3

        def liss(b, _):
            def ibody(jj, _):
                j = b * 8 + jj
                pltpu.make_async_copy(
                    labels_hbm.at[pl.ds(pl.multiple_of(rb8, 8), 8),
                                  pl.ds((tile0 + j) * 128, 128)],
                    lbuf.at[(b & 1) * 8 + jj], sem).start()
                return 0
            lax.fori_loop(0, jnp.minimum(jnp.int32(8), ntf - b * 8), ibody, 0)
            return 0

        def ldrain_mask(b, _):
            nb = jnp.minimum(jnp.int32(8), ntf - b * 8)
            def dbody(jj, _):
                j = b * 8 + jj
                pltpu.make_async_copy(
                    labels_hbm.at[pl.ds(pl.multiple_of(rb8, 8), 8),
                                  pl.ds((tile0 + j) * 128, 128)],
                    lbuf.at[(b & 1) * 8 + jj], sem).wait()
                return 0
            lax.fori_loop(0, nb, dbody, 0)
            @pl.when(b + 1 < nbatch)
            def _():
                liss(b + 1, 0)
            def mbody(p, _):
                jj = p >> 3
                rr = p & 7
                j = b * 8 + jj
                for t in range(8):
                    lab = lbuf[(b & 1) * 8 + jj, rr, pl.ds(t * 16, 16)]
                    d = slab[j, rr, pl.ds(t * 16, 16)]
                    slab[j, rr, pl.ds(t * 16, 16)] = (
                        jnp.where(lab == 1.0, d, ninf))
                return 0
            lax.fori_loop(0, nb * 8, mbody, 0)
            return 0

        liss(0, 0)
        lax.fori_loop(0, nbatch, ldrain_mask, 0)

        @pl.when(has_tail)
        def _():
            pltpu.make_async_copy(
                ltail_hbm.at[pl.ds(pl.multiple_of(rb8, 8), 8)], lbuf.at[0], sem).start()
            pltpu.make_async_copy(
                ltail_hbm.at[pl.ds(pl.multiple_of(rb8, 8), 8)], lbuf.at[0], sem).wait()
            for rr in range(8):
                for t in range(8):
                    lab = lbuf[0, rr, pl.ds(t * 16, 16)]
                    d = slab[ntf, rr, pl.ds(t * 16, 16)]
                    slab[ntf, rr, pl.ds(t * 16, 16)] = (
                        jnp.where(lab == 1.0, d, ninf))

        def p2(r, _):
            cacc = build_hier(r, ntt, True)
            cnt_r = _rsum(cacc)
            candc[...] = candc[...] + jnp.where(iota == r, cnt_r, 0.0)
            vals, idxs = extract10(r, ntt, tile0)
            rv = candv[pl.ds(r * 64 + 16, 16)]
            ri = candi[pl.ds(r * 64 + 16, 16)]
            mv, mi = rank_merge_refs(rv, ri, vals, idxs)
            candv[pl.ds(r * 64 + 16, 16)] = mv
            candi[pl.ds(r * 64 + 16, 16)] = mi
            return 0
        lax.fori_loop(0, 8, p2, 0)
        return 0

    lax.fori_loop(0, _NSEG, seg_body, 0)

    _mo = lambda x: pl.multiple_of(x, 8)
    pltpu.sync_copy(candv, shv.at[pl.ds(_mo(sid * 544), 512)])
    pltpu.sync_copy(candc, shv.at[pl.ds(_mo(sid * 544 + 512), 16)])
    pltpu.sync_copy(candi, shi.at[pl.ds(_mo(sid * 512), 512)])
    plsc.subcore_barrier()
    pair = sid & ~1
    pltpu.sync_copy(shv.at[pl.ds(_mo(pair * 544), 528)], rdav)
    pltpu.sync_copy(shi.at[pl.ds(_mo(pair * 512), 512)], rdai)
    pltpu.sync_copy(shv.at[pl.ds(_mo((pair + 1) * 544), 528)], rdbv)
    pltpu.sync_copy(shi.at[pl.ds(_mo((pair + 1) * 512), 512)], rdbi)

    cnt_both = rdav[pl.ds(512, 16)] + rdbv[pl.ds(512, 16)]

    def loss_row(r, acc):
        a1, b1, a2, b2 = acc
        pv, pi = rank_merge_refs(rdav[pl.ds(r * 64, 16)],
                                 rdai[pl.ds(r * 64, 16)],
                                 rdbv[pl.ds(r * 64, 16)],
                                 rdbi[pl.ds(r * 64, 16)])
        tv, ti = rank_merge_refs(rdav[pl.ds(r * 64 + 16, 16)],
                                 rdai[pl.ds(r * 64 + 16, 16)],
                                 rdbv[pl.ds(r * 64 + 16, 16)],
                                 rdbi[pl.ds(r * 64 + 16, 16)])
        num_true = _rsum(jnp.where(iota == r, cnt_both, 0.0))
        nv = jnp.minimum(jnp.int32(_K), num_true.astype(jnp.int32))
        pnt_i = jnp.where(iota < _K, jnp.int32(1), jnp.int32(0))
        for j in range(_K):
            jvec = iota * 0 + j
            keep_i = (jnp.where(pi != ti[j], jnp.int32(1), jnp.int32(0))
                      + jnp.where(jvec >= nv, jnp.int32(1), jnp.int32(0)))
            pnt_i = jnp.where(keep_i > 0, pnt_i, jnp.int32(0))
        npnt = _rsum(pnt_i)
        s1 = jnp.float32(0.0)
        s2 = jnp.float32(0.0)
        for i in range(_K):
            ti_v = tv[i]
            ivec = iota * 0 + i
            one = jnp.int32(1)
            zero = jnp.int32(0)
            validi_i = jnp.where(ivec < nv, one, zero)
            tvalid_i = jnp.where(iota < nv, one, zero)
            gti_i = jnp.where(iota > i, one, zero)
            c1 = jnp.maximum(jnp.float32(_DELTA) - (ti_v - tv), 0.0)
            m1_i = gti_i * tvalid_i * validi_i
            s1 = s1 + _rsum(jnp.where(m1_i > 0, c1, 0.0))
            c2 = jnp.maximum(jnp.float32(_DELTA) - (ti_v - pv), 0.0)
            m2_i = pnt_i * validi_i
            s2 = s2 + _rsum(jnp.where(m2_i > 0, c2, 0.0))
        cnt1 = ((nv * (nv - 1)) >> 1).astype(jnp.float32)
        cnt2 = (nv * npnt).astype(jnp.float32)
        a1 = a1 + jnp.where(iota == r, s1, 0.0)
        b1 = b1 + jnp.where(iota == r, cnt1, 0.0)
        a2 = a2 + jnp.where(iota == r, s2, 0.0)
        b2 = b2 + jnp.where(iota == r, cnt2, 0.0)
        return a1, b1, a2, b2

    zf = jnp.zeros((16,), jnp.float32)
    a1, b1, a2, b2 = lax.fori_loop(0, 8, loss_row, (zf, zf, zf, zf))
    evenf = jnp.where(is_b == 0, jnp.float32(1.0), jnp.float32(0.0))
    ovec[pl.ds(0, 16)] = a1 * evenf
    ovec[pl.ds(16, 16)] = b1 * evenf
    ovec[pl.ds(32, 16)] = a2 * evenf
    ovec[pl.ds(48, 16)] = b2 * evenf
    pltpu.sync_copy(ovec, out_hbm.at[pl.ds(wid * 64, 64)])


def _red_body(x_ref, o_ref):
    x = x_ref[...]
    s1 = x[:, 0:16]
    c1 = x[:, 16:32]
    s2 = x[:, 32:48]
    c2 = x[:, 48:64]
    l = s1 / jnp.maximum(c1, 1.0) + s2 / jnp.maximum(c2, 1.0)
    o_ref[...] = (jnp.sum(l) * jnp.float32(1.0 / _B)).reshape(1, 1)


_tc_reduce = pl.pallas_call(
    _red_body,
    out_shape=jax.ShapeDtypeStruct((1, 1), jnp.float32),
)


@jax.jit
def kernel(scores, true_labels):
    stail = jnp.pad(scores[:, 781 * 128:], ((0, 0), (0, 128 - _TAILC)),
                    constant_values=-jnp.inf)
    ltail = jnp.pad(true_labels[:, 781 * 128:], ((0, 0), (0, 128 - _TAILC)),
                    constant_values=0.0)
    partial = _sc_loss(scores, true_labels, stail, ltail)
    return _tc_reduce(partial.reshape(_NW, 64))[0, 0]


# speedup vs baseline: 1.0122x; 1.0087x over previous
import functools
import jax
import jax.numpy as jnp
from jax import lax
from jax.experimental import pallas as pl
from jax.experimental.pallas import tpu as pltpu
from jax.experimental.pallas import tpu_sc as plsc

_B = 128
_V = 100000
_K = 10
_DELTA = 0.01
_NC = 2
_NS = 16
_NW = _NC * _NS
_NTILES = 782
_NFULL_A = 391
_NFULL_B = 390
_SEG = 98
_NSEG = 4
_TAILC = _V - 781 * 128
_BIG = 2147483647

_mesh = plsc.VectorSubcoreMesh(
    core_axis_name="c", subcore_axis_name="s", num_cores=_NC, num_subcores=_NS
)


@functools.partial(
    pl.kernel,
    out_type=jax.ShapeDtypeStruct((_NW * 64,), jnp.float32),
    mesh=_mesh,
    scratch_types=[
        pltpu.VMEM((_SEG, 8, 128), jnp.float32),
        pltpu.VMEM((112 * 16,), jnp.float32),
        pltpu.VMEM((112 * 16,), jnp.int32),
        pltpu.VMEM((7 * 16,), jnp.float32),
        pltpu.VMEM((7 * 16,), jnp.int32),
        pltpu.VMEM((16, 8, 128), jnp.float32),
        pltpu.VMEM((512,), jnp.float32),
        pltpu.VMEM((512,), jnp.int32),
        pltpu.VMEM((16,), jnp.float32),
        pltpu.VMEM((528,), jnp.float32),
        pltpu.VMEM((512,), jnp.int32),
        pltpu.VMEM((528,), jnp.float32),
        pltpu.VMEM((512,), jnp.int32),
        pltpu.VMEM((16,), jnp.float32),
        pltpu.VMEM((16,), jnp.int32),
        pltpu.VMEM((64,), jnp.float32),
        pltpu.VMEM_SHARED((16 * 544,), jnp.float32),
        pltpu.VMEM_SHARED((16 * 512,), jnp.int32),
        pltpu.SemaphoreType.DMA,
    ],
)
def _sc_loss(scores_hbm, labels_hbm, stail_hbm, ltail_hbm, out_hbm, slab,
             gmval, gme, sgval, sge, lbuf, candv, candi, candc, rdav, rdai,
             rdbv, rdbi, tmpv, tmpi, ovec, shv, shi, sem):
    cid = lax.axis_index("c")
    sid = lax.axis_index("s")
    wid = cid * _NS + sid
    iota = lax.iota(jnp.int32, 16)
    ninf = jnp.float32(-jnp.inf)
    ninf_vec = jnp.full((16,), ninf, jnp.float32)
    big_vec = jnp.full((16,), _BIG, jnp.int32)

    is_b = sid & 1
    rb8 = (cid * 8 + (sid >> 1)) * 8
    nfull = jnp.where(is_b == 1, jnp.int32(_NFULL_B), jnp.int32(_NFULL_A))
    half0 = jnp.where(is_b == 1, jnp.int32(_NFULL_A), jnp.int32(0))

    _dnums = lax.GatherDimensionNumbers(
        offset_dims=(), collapsed_slice_dims=(0,), start_index_map=(0,))

    def _gather16(v, idx):
        return lax.gather(v, idx[:, None], _dnums, slice_sizes=(1,),
                          mode=lax.GatherScatterMode.PROMISE_IN_BOUNDS)

    def _bfly(v, op):
        for sh in (8, 4, 2, 1):
            v = op(v, _gather16(v, iota ^ sh))
        return v[0]

    def _rmax_f(v):
        return _bfly(v, jnp.maximum)

    def _rmin_i(v):
        return _bfly(v, jnp.minimum)

    def _rsum(v):
        return _bfly(v, lax.add)

    def pair_tree(vs, ts):
        vs, ts = list(vs), list(ts)
        while len(vs) > 1:
            nv, nt = [], []
            for u in range(0, len(vs) - 1, 2):
                gt = vs[u + 1] > vs[u]
                nv.append(jnp.where(gt, vs[u + 1], vs[u]))
                nt.append(jnp.where(gt, ts[u + 1], ts[u]))
            if len(vs) % 2:
                nv.append(vs[-1])
                nt.append(ts[-1])
            vs, ts = nv, nt
        return vs[0], ts[0]

    def tile_tree(j2, r):
        vs = [slab[j2, r, pl.ds(t * 16, 16)] for t in range(8)]
        ts = [jnp.full((16,), t, jnp.int32) for t in range(8)]
        while len(vs) > 1:
            nv, nt = [], []
            for u in range(0, len(vs), 2):
                gt = vs[u + 1] > vs[u]
                nv.append(jnp.where(gt, vs[u + 1], vs[u]))
                nt.append(jnp.where(gt, ts[u + 1], ts[u]))
            vs, ts = nv, nt
        return vs[0], ts[0] * 16 + iota

    def build_hier(r, ntt, count_finite):
        def bpad(j, _):
            gmval[pl.ds(j * 16, 16)] = ninf_vec
            gme[pl.ds(j * 16, 16)] = big_vec
            return 0
        lax.fori_loop(ntt, 112, bpad, 0)

        def bt(j2, cacc):
            gv, gt3 = tile_tree(j2, r)
            gmval[pl.ds(j2 * 16, 16)] = gv
            gme[pl.ds(j2 * 16, 16)] = j2 * 128 + gt3
            if count_finite:
                for t in range(8):
                    d = slab[j2, r, pl.ds(t * 16, 16)]
                    cacc = cacc + jnp.where(d > ninf, 1.0, 0.0)
            return cacc
        cacc = lax.fori_loop(0, ntt, bt, jnp.zeros((16,), jnp.float32))

        def bs(s, _):
            vv = [gmval[pl.ds(s * 256 + tt * 16, 16)] for tt in range(16)]
            ii = [gme[pl.ds(s * 256 + tt * 16, 16)] for tt in range(16)]
            sv, si = pair_tree(vv, ii)
            sgval[pl.ds(s * 16, 16)] = sv
            sge[pl.ds(s * 16, 16)] = si
            return 0
        lax.fori_loop(0, 7, bs, 0)
        return cacc

    def extract10(r, ntt, tile0):
        def round_body(rnd, carry):
            vals, idxs = carry
            m, mi = pair_tree(
                [sgval[pl.ds(s * 16, 16)] for s in range(7)],
                [sge[pl.ds(s * 16, 16)] for s in range(7)])
            for sh in (8, 4, 2, 1):
                gv_ = _gather16(m, iota ^ sh)
                ge_ = _gather16(mi, iota ^ sh)
                upd = (gv_ > m) | ((gv_ == m) & (ge_ < mi))
                m = jnp.where(upd, gv_, m)
                mi = jnp.where(upd, ge_, mi)
            mval = m[0]
            e = jnp.minimum(mi[0], ntt * 128 - 1)
            g = jnp.minimum(tile0 * 128 + e, jnp.int32(_V - 1))
            vals = jnp.where(iota == rnd, mval, vals)
            idxs = jnp.where(iota == rnd, g, idxs)
            j2 = e >> 7
            d = slab[j2, r, pl.ds(((e >> 4) & 7) * 16, 16)]
            slab[j2, r, pl.ds(((e >> 4) & 7) * 16, 16)] = (
                jnp.where(iota == (e & 15), ninf, d))
            gv, gt3 = tile_tree(j2, r)
            gmval[pl.ds(j2 * 16, 16)] = gv
            gme[pl.ds(j2 * 16, 16)] = j2 * 128 + gt3
            s2 = j2 >> 4
            sv, si = pair_tree(
                [gmval[pl.ds(s2 * 256 + tt * 16, 16)] for tt in range(16)],
                [gme[pl.ds(s2 * 256 + tt * 16, 16)] for tt in range(16)])
            sgval[pl.ds(s2 * 16, 16)] = sv
            sge[pl.ds(s2 * 16, 16)] = si
            return vals, idxs
        return lax.fori_loop(0, _K, round_body,
                             (ninf_vec, jnp.zeros((16,), jnp.int32)))

    def restore10(r, tile0, vals, idxs):
        def rbody(r2, _):
            mval = _rmax_f(jnp.where(iota == r2, vals, ninf_vec))
            g = _rmin_i(jnp.where(iota == r2, idxs, big_vec))
            e = g - tile0 * 128
            j2 = e >> 7
            d = slab[j2, r, pl.ds(((e >> 4) & 7) * 16, 16)]
            slab[j2, r, pl.ds(((e >> 4) & 7) * 16, 16)] = (
                jnp.where(iota == (e & 15), mval, d))
            return 0
        lax.fori_loop(0, _K, rbody, 0)

    def rank_merge_refs(rv, ri, sv, si):
        cr = jnp.zeros((16,), jnp.int32)
        cs = jnp.zeros((16,), jnp.int32)
        for j in range(_K):
            cr = cr + jnp.where(sv[j] > rv, 1, 0)
            cs = cs + jnp.where(rv[j] >= sv, 1, 0)
        rank_r = iota + cr
        rank_s = iota + cs
        src_a = jnp.zeros((16,), jnp.int32)
        has_a = jnp.zeros((16,), jnp.int32)
        src_b = jnp.zeros((16,), jnp.int32)
        for j in range(_K):
            ra = rank_r[j]
            src_a = src_a + jnp.where(iota == ra, j, 0)
            has_a = has_a + jnp.where(iota == ra, 1, 0)
            rb = rank_s[j]
            src_b = src_b + jnp.where(iota == rb, j, 0)
        va = _gather16(rv, src_a)
        ia = _gather16(ri, src_a)
        vb = _gather16(sv, src_b)
        ib = _gather16(si, src_b)
        mv = jnp.where(has_a > 0, va, vb)
        mi = jnp.where(has_a > 0, ia, ib)
        return mv, mi

    def seg_params(seg):
        tile0_rel = seg * _SEG
        ntf = jnp.minimum(jnp.int32(_SEG), nfull - tile0_rel)
        has_tail = (is_b == 1) & (seg == 3)
        ntt = ntf + jnp.where(has_tail, 1, 0)
        tile0 = half0 + tile0_rel
        return tile0, ntf, ntt, has_tail

    def cinit(i, _):
        candv[pl.ds(i * 16, 16)] = ninf_vec
        candi[pl.ds(i * 16, 16)] = big_vec
        return 0
    lax.fori_loop(0, 32, cinit, 0)
    candc[...] = jnp.zeros((16,), jnp.float32)

    def seg_body(seg, _):
        tile0, ntf, ntt, has_tail = seg_params(seg)

        def siss(j, _):
            pltpu.make_async_copy(
                scores_hbm.at[pl.ds(pl.multiple_of(rb8, 8), 8), pl.ds((tile0 + j) * 128, 128)],
                slab.at[j], sem).start()
            return 0
        lax.fori_loop(0, ntf, siss, 0)

        @pl.when(has_tail)
        def _():
            pltpu.make_async_copy(
                stail_hbm.at[pl.ds(pl.multiple_of(rb8, 8), 8)], slab.at[ntf], sem).start()

        def sdrain(j, _):
            pltpu.make_async_copy(
                scores_hbm.at[pl.ds(pl.multiple_of(rb8, 8), 8), pl.ds((tile0 + j) * 128, 128)],
                slab.at[j], sem).wait()
            return 0
        lax.fori_loop(0, ntf, sdrain, 0)

        @pl.when(has_tail)
        def _():
            pltpu.make_async_copy(
                stail_hbm.at[pl.ds(pl.multiple_of(rb8, 8), 8)], slab.at[ntf], sem).wait()

        nbatch = (ntf + 7) >> 3

        def liss(b, _):
            def ibody(jj, _):
                j = b * 8 + jj
                pltpu.make_async_copy(
                    labels_hbm.at[pl.ds(pl.multiple_of(rb8, 8), 8),
                                  pl.ds((tile0 + j) * 128, 128)],
                    lbuf.at[(b & 1) * 8 + jj], sem).start()
                return 0
            lax.fori_loop(0, jnp.minimum(jnp.int32(8), ntf - b * 8), ibody, 0)
            return 0

        liss(0, 0)

        def p1(r, _):
            build_hier(r, ntt, False)
            vals, idxs = extract10(r, ntt, tile0)
            restore10(r, tile0, vals, idxs)
            rv = candv[pl.ds(r * 64, 16)]
            ri = candi[pl.ds(r * 64, 16)]
            mv, mi = rank_merge_refs(rv, ri, vals, idxs)
            candv[pl.ds(r * 64, 16)] = mv
            candi[pl.ds(r * 64, 16)] = mi
            return 0
        lax.fori_loop(0, 8, p1, 0)

        def ldrain_mask(b, _):
            nb = jnp.minimum(jnp.int32(8), ntf - b * 8)
            def dbody(jj, _):
                j = b * 8 + jj
                pltpu.make_async_copy(
                    labels_hbm.at[pl.ds(pl.multiple_of(rb8, 8), 8),
                                  pl.ds((tile0 + j) * 128, 128)],
                    lbuf.at[(b & 1) * 8 + jj], sem).wait()
                return 0
            lax.fori_loop(0, nb, dbody, 0)
            @pl.when(b + 1 < nbatch)
            def _():
                liss(b + 1, 0)
            def mbody(p, _):
                jj = p >> 3
                rr = p & 7
                j = b * 8 + jj
                for t in range(8):
                    lab = lbuf[(b & 1) * 8 + jj, rr, pl.ds(t * 16, 16)]
                    d = slab[j, rr, pl.ds(t * 16, 16)]
                    slab[j, rr, pl.ds(t * 16, 16)] = (
                        jnp.where(lab == 1.0, d, ninf))
                return 0
            lax.fori_loop(0, nb * 8, mbody, 0)
            return 0

        lax.fori_loop(0, nbatch, ldrain_mask, 0)

        @pl.when(has_tail)
        def _():
            pltpu.make_async_copy(
                ltail_hbm.at[pl.ds(pl.multiple_of(rb8, 8), 8)], lbuf.at[0], sem).start()
            pltpu.make_async_copy(
                ltail_hbm.at[pl.ds(pl.multiple_of(rb8, 8), 8)], lbuf.at[0], sem).wait()
            for rr in range(8):
                for t in range(8):
                    lab = lbuf[0, rr, pl.ds(t * 16, 16)]
                    d = slab[ntf, rr, pl.ds(t * 16, 16)]
                    slab[ntf, rr, pl.ds(t * 16, 16)] = (
                        jnp.where(lab == 1.0, d, ninf))

        def p2(r, _):
            cacc = build_hier(r, ntt, True)
            cnt_r = _rsum(cacc)
            candc[...] = candc[...] + jnp.where(iota == r, cnt_r, 0.0)
            vals, idxs = extract10(r, ntt, tile0)
            rv = candv[pl.ds(r * 64 + 16, 16)]
            ri = candi[pl.ds(r * 64 + 16, 16)]
            mv, mi = rank_merge_refs(rv, ri, vals, idxs)
            candv[pl.ds(r * 64 + 16, 16)] = mv
            candi[pl.ds(r * 64 + 16, 16)] = mi
            return 0
        lax.fori_loop(0, 8, p2, 0)
        return 0

    lax.fori_loop(0, _NSEG, seg_body, 0)

    _mo = lambda x: pl.multiple_of(x, 8)
    pltpu.sync_copy(candv, shv.at[pl.ds(_mo(sid * 544), 512)])
    pltpu.sync_copy(candc, shv.at[pl.ds(_mo(sid * 544 + 512), 16)])
    pltpu.sync_copy(candi, shi.at[pl.ds(_mo(sid * 512), 512)])
    plsc.subcore_barrier()
    pair = sid & ~1
    pltpu.sync_copy(shv.at[pl.ds(_mo(pair * 544), 528)], rdav)
    pltpu.sync_copy(shi.at[pl.ds(_mo(pair * 512), 512)], rdai)
    pltpu.sync_copy(shv.at[pl.ds(_mo((pair + 1) * 544), 528)], rdbv)
    pltpu.sync_copy(shi.at[pl.ds(_mo((pair + 1) * 512), 512)], rdbi)

    cnt_both = rdav[pl.ds(512, 16)] + rdbv[pl.ds(512, 16)]

    def loss_row(r, acc):
        a1, b1, a2, b2 = acc
        pv, pi = rank_merge_refs(rdav[pl.ds(r * 64, 16)],
                                 rdai[pl.ds(r * 64, 16)],
                                 rdbv[pl.ds(r * 64, 16)],
                                 rdbi[pl.ds(r * 64, 16)])
        tv, ti = rank_merge_refs(rdav[pl.ds(r * 64 + 16, 16)],
                                 rdai[pl.ds(r * 64 + 16, 16)],
                                 rdbv[pl.ds(r * 64 + 16, 16)],
                                 rdbi[pl.ds(r * 64 + 16, 16)])
        num_true = _rsum(jnp.where(iota == r, cnt_both, 0.0))
        nv = jnp.minimum(jnp.int32(_K), num_true.astype(jnp.int32))
        pnt_i = jnp.where(iota < _K, jnp.int32(1), jnp.int32(0))
        for j in range(_K):
            jvec = iota * 0 + j
            keep_i = (jnp.where(pi != ti[j], jnp.int32(1), jnp.int32(0))
                      + jnp.where(jvec >= nv, jnp.int32(1), jnp.int32(0)))
            pnt_i = jnp.where(keep_i > 0, pnt_i, jnp.int32(0))
        npnt = _rsum(pnt_i)
        s1 = jnp.float32(0.0)
        s2 = jnp.float32(0.0)
        for i in range(_K):
            ti_v = tv[i]
            ivec = iota * 0 + i
            one = jnp.int32(1)
            zero = jnp.int32(0)
            validi_i = jnp.where(ivec < nv, one, zero)
            tvalid_i = jnp.where(iota < nv, one, zero)
            gti_i = jnp.where(iota > i, one, zero)
            c1 = jnp.maximum(jnp.float32(_DELTA) - (ti_v - tv), 0.0)
            m1_i = gti_i * tvalid_i * validi_i
            s1 = s1 + _rsum(jnp.where(m1_i > 0, c1, 0.0))
            c2 = jnp.maximum(jnp.float32(_DELTA) - (ti_v - pv), 0.0)
            m2_i = pnt_i * validi_i
            s2 = s2 + _rsum(jnp.where(m2_i > 0, c2, 0.0))
        cnt1 = ((nv * (nv - 1)) >> 1).astype(jnp.float32)
        cnt2 = (nv * npnt).astype(jnp.float32)
        a1 = a1 + jnp.where(iota == r, s1, 0.0)
        b1 = b1 + jnp.where(iota == r, cnt1, 0.0)
        a2 = a2 + jnp.where(iota == r, s2, 0.0)
        b2 = b2 + jnp.where(iota == r, cnt2, 0.0)
        return a1, b1, a2, b2

    zf = jnp.zeros((16,), jnp.float32)
    a1, b1, a2, b2 = lax.fori_loop(0, 8, loss_row, (zf, zf, zf, zf))
    evenf = jnp.where(is_b == 0, jnp.float32(1.0), jnp.float32(0.0))
    ovec[pl.ds(0, 16)] = a1 * evenf
    ovec[pl.ds(16, 16)] = b1 * evenf
    ovec[pl.ds(32, 16)] = a2 * evenf
    ovec[pl.ds(48, 16)] = b2 * evenf
    pltpu.sync_copy(ovec, out_hbm.at[pl.ds(wid * 64, 64)])


def _red_body(x_ref, o_ref):
    x = x_ref[...]
    s1 = x[:, 0:16]
    c1 = x[:, 16:32]
    s2 = x[:, 32:48]
    c2 = x[:, 48:64]
    l = s1 / jnp.maximum(c1, 1.0) + s2 / jnp.maximum(c2, 1.0)
    o_ref[...] = (jnp.sum(l) * jnp.float32(1.0 / _B)).reshape(1, 1)


_tc_reduce = pl.pallas_call(
    _red_body,
    out_shape=jax.ShapeDtypeStruct((1, 1), jnp.float32),
)


@jax.jit
def kernel(scores, true_labels):
    stail = jnp.pad(scores[:, 781 * 128:], ((0, 0), (0, 128 - _TAILC)),
                    constant_values=-jnp.inf)
    ltail = jnp.pad(true_labels[:, 781 * 128:], ((0, 0), (0, 128 - _TAILC)),
                    constant_values=0.0)
    partial = _sc_loss(scores, true_labels, stail, ltail)
    return _tc_reduce(partial.reshape(_NW, 64))[0, 0]
